# baseline (speedup 1.0000x reference)
import functools

import jax
import jax.numpy as jnp
from jax import lax
from jax.experimental import pallas as pl
from jax.experimental.pallas import tpu as pltpu
from jax.experimental.pallas import tpu_sc as plsc

NW, NT, ND, D = 50000, 10000, 10000, 128
E_WW, E = 200000, 100000
LANES = 16
NDP = 10240
NWP = 51200


def _linear_chain_kernel(nmm, slices, x_ref, *refs):
    ws = refs[:2 * nmm]
    outs = refs[2 * nmm:]
    y = x_ref[...]
    for i in range(nmm):
        w = ws[2 * i][...]
        b = ws[2 * i + 1][...]
        y = lax.dot_general(y, w, (((1,), (1,)), ((), ())),
                            preferred_element_type=jnp.float32) + b
    for r, (start, nc) in zip(outs, slices):
        r[...] = y[:, start:start + nc]


def _linear_chain(x, wbs, out_slices, blk):
    n = x.shape[0]
    nmm = len(wbs) // 2
    grid = (n // blk,)
    w_spec = pl.BlockSpec((D, D), lambda i: (0, 0))
    b_spec = pl.BlockSpec((1, D), lambda i: (0, 0))
    in_specs = [pl.BlockSpec((blk, D), lambda i: (i, 0))]
    for _ in range(nmm):
        in_specs += [w_spec, b_spec]
    out_shapes = [jax.ShapeDtypeStruct((n, c), jnp.float32)
                  for _, c in out_slices]
    out_specs = [pl.BlockSpec((blk, c), lambda i: (i, 0))
                 for _, c in out_slices]
    return pl.pallas_call(
        functools.partial(_linear_chain_kernel, nmm, out_slices),
        grid=grid,
        in_specs=in_specs,
        out_specs=out_specs,
        out_shape=out_shapes,
    )(x, *wbs)


_SC_PARAMS = pltpu.CompilerParams(needs_layout_passes=False,
                                  use_tc_tiling_on_sc=False)


def _full16(v, dtype=jnp.int32):
    return jnp.full((LANES,), v, dtype=dtype)


def _splat(vec, m):
    return jnp.take_along_axis(vec, _full16(m), axis=0)


def _make_counts(ndpad, nb, bl):
    r_hist = ndpad // 16
    per_tile = ndpad // 16
    gpb = bl // 16
    mesh = plsc.VectorSubcoreMesh(core_axis_name="c", subcore_axis_name="s")
    scratch = [
        pltpu.VMEM((nb, bl), jnp.int32),
        pltpu.VMEM((r_hist * 16,), jnp.float32),
        pltpu.VMEM((128, 16), jnp.float32),
        pltpu.VMEM((1, 128), jnp.int32),
        pltpu.VMEM((8, 16), jnp.float32),
        pltpu.VMEM((1, 128), jnp.float32),
        pltpu.VMEM_SHARED((r_hist, 16), jnp.float32),
    ]
    out_type = jax.ShapeDtypeStruct((2 * ndpad,), jnp.float32)

    @functools.partial(pl.kernel, out_type=out_type, mesh=mesh,
                       scratch_types=scratch, compiler_params=_SC_PARAMS)
    def counts(didx, out, db, hist, hstage, idc, cb, invs, cnt):
        c = lax.axis_index("c")
        s = lax.axis_index("s")
        iota = lax.iota(jnp.int32, LANES)
        ones = jnp.full((LANES,), 1.0, jnp.float32)
        zeros = jnp.zeros((LANES,), jnp.float32)

        pltpu.sync_copy(didx.at[s], db)

        def zh(r, _):
            hist[pl.ds(r * 16, 16)] = zeros
            return _
        lax.fori_loop(0, r_hist, zh, 0)
        for r in range(8):
            cb[r, pl.ds(0, 16)] = zeros
        crow0 = s * (r_hist // 16)

        def zc(i, _):
            pltpu.sync_copy(cb, cnt.at[pl.ds(crow0 + 8 * i, 8)])
            return _
        lax.fori_loop(0, r_hist // 128, zc, 0)
        plsc.subcore_barrier()

        def per_group(g, _):
            def pg2(jj, _):
                d16v = db[g, pl.ds(jj * 16, 16)]
                for m in range(16):
                    dsp = _splat(d16v, m)
                    plsc.addupdate_scatter(hist, [dsp], ones, mask=iota == 0)
                return _
            lax.fori_loop(0, gpb, pg2, 0)
            return _
        lax.fori_loop(0, nb, per_group, 0)
        plsc.subcore_barrier()

        def merge(t, _):
            base = t * 128
            for m in range(8):
                idc[0, pl.ds(m * 16, 16)] = _full16(base + m * 16) + iota

            def stg(q, _):
                hstage[q, pl.ds(0, 16)] = hist[pl.ds((base + q) * 16, 16)]
                return _
            lax.fori_loop(0, 128, stg, 0)
            pltpu.sync_copy(hstage, cnt.at[idc.at[0]], add=True)
            return _
        lax.fori_loop(0, r_hist // 128, merge, 0)
        plsc.subcore_barrier()

        row0 = s * per_tile
        obase = c * ndpad + row0

        def fin(i, _):
            r = row0 + 128 * i
            pltpu.sync_copy(cnt.at[pl.ds(lax.shift_right_logical(r, 4), 8)],
                            cb)
            for q in range(8):
                cv = cb[q, pl.ds(0, 16)]
                inv = jnp.where(cv > 0.0, 1.0 / jnp.maximum(cv, 1.0), 0.0)
                invs[0, pl.ds(q * 16, 16)] = inv
            pltpu.sync_copy(invs.at[0], out.at[pl.ds(obase + 128 * i, 128)])
            return _
        lax.fori_loop(0, per_tile // 128, fin, 0)

    return counts


def _make_seg(drow, ndpad, nb, bl, tile_stride, src_off, inv_from_hbm,
              nsrc, nesets, nout, phases):
    r_hist = ndpad // 16
    per_tile = ndpad // 16
    assert per_tile % 128 == 0
    mesh = plsc.VectorSubcoreMesh(core_axis_name="c", subcore_axis_name="s")
    kch = drow // 16
    has_prev = any(ph["prev"] is not None for ph in phases)

    gpb = bl // 16
    scratch = [
        pltpu.VMEM((nb, bl), jnp.int32),
        pltpu.VMEM((nb, bl), jnp.int32),
        pltpu.VMEM((nb, bl), jnp.float32),
        pltpu.VMEM((bl, drow), jnp.float32),
        pltpu.VMEM((bl, drow), jnp.float32),
        pltpu.VMEM((128, drow), jnp.float32),
        pltpu.VMEM((128, drow), jnp.float32),
        pltpu.SemaphoreType.DMA,
        pltpu.SemaphoreType.DMA,
        pltpu.SemaphoreType.DMA,
    ]
    if inv_from_hbm:
        scratch.append(pltpu.VMEM((1, 128), jnp.float32))
    else:
        scratch += [
            pltpu.VMEM((r_hist * 16,), jnp.float32),
            pltpu.VMEM((128, 16), jnp.float32),
            pltpu.VMEM((1, 128), jnp.int32),
            pltpu.VMEM((8, 16), jnp.float32),
        ]
    scratch.append(pltpu.VMEM_SHARED((ndpad, drow), jnp.float32))
    if not inv_from_hbm:
        scratch.append(pltpu.VMEM_SHARED((r_hist, 16), jnp.float32))

    out_type = [jax.ShapeDtypeStruct((2 * ndpad, drow), jnp.float32)
                for _ in range(nout)]

    @functools.partial(pl.kernel, out_type=out_type, mesh=mesh,
                       scratch_types=scratch, compiler_params=_SC_PARAMS)
    def seg(*args):
        it = iter(args)
        srcs = [next(it) for _ in range(nsrc)]
        esets = [(next(it), next(it), next(it)) for _ in range(nesets)]
        invh = next(it) if inv_from_hbm else None
        outs = [next(it) for _ in range(nout)]
        sb, db, wb, msgin, msgout, sumb, ob = (next(it) for _ in range(7))
        gsem = next(it)
        ssem = next(it)
        next(it)
        if inv_from_hbm:
            invb = next(it)
        else:
            hist, hstage, idc, cb = (next(it) for _ in range(4))
        acc = next(it)
        cnt = None if inv_from_hbm else next(it)

        c = lax.axis_index("c")
        s = lax.axis_index("s")
        iota = lax.iota(jnp.int32, LANES)
        ones = jnp.full((LANES,), 1.0, jnp.float32)
        zeros = jnp.zeros((LANES,), jnp.float32)

        tid = c * tile_stride + s
        row0 = s * per_tile
        obase0 = c * ndpad + row0

        def preload(eset):
            sidx, didx, ew = esets[eset]
            pltpu.sync_copy(sidx.at[tid], sb)
            pltpu.sync_copy(didx.at[tid], db)
            pltpu.sync_copy(ew.at[tid], wb)
            if src_off:
                off = _full16(c * src_off)

                def adj(g, _):
                    def adj2(k, _):
                        sb[g, pl.ds(k * 16, 16)] = (sb[g, pl.ds(k * 16, 16)]
                                                    + off)
                        return _
                    lax.fori_loop(0, gpb, adj2, 0)
                    return _
                lax.fori_loop(0, nb, adj, 0)

        cur_eset = None
        for p, ph in enumerate(phases):
            do_hist = (not inv_from_hbm) and ph["hist"]
            if ph["eset"] != cur_eset:
                preload(ph["eset"])
                cur_eset = ph["eset"]

            def zz(r, _):
                for k in range(kch):
                    sumb[r, pl.ds(k * 16, 16)] = zeros
                return _
            lax.fori_loop(0, 128, zz, 0)

            def za(i, _):
                pltpu.sync_copy(sumb, acc.at[pl.ds(row0 + 128 * i, 128)])
                return _
            lax.fori_loop(0, per_tile // 128, za, 0)
            if do_hist:
                def zh(r, _):
                    hist[pl.ds(r * 16, 16)] = zeros
                    return _
                lax.fori_loop(0, r_hist, zh, 0)
                for r in range(8):
                    cb[r, pl.ds(0, 16)] = zeros
                crow0 = s * (r_hist // 16)

                def zc(i, _):
                    pltpu.sync_copy(cb, cnt.at[pl.ds(crow0 + 8 * i, 8)])
                    return _
                lax.fori_loop(0, r_hist // 128, zc, 0)
            plsc.subcore_barrier()

            if do_hist:
                def histloop(g, _):
                    def histg(jj, _):
                        d16v = db[g, pl.ds(jj * 16, 16)]
                        for m in range(16):
                            dsp = _splat(d16v, m)
                            plsc.addupdate_scatter(hist, [dsp], ones,
                                                   mask=iota == 0)
                        return _
                    lax.fori_loop(0, gpb, histg, 0)
                    return _
                lax.fori_loop(0, nb, histloop, 0)

            src_p = srcs[ph["src"]]
            pltpu.async_copy(src_p.at[sb.at[0]], msgin, gsem)

            def edge_block(g, _):
                pltpu.make_async_copy(src_p.at[sb.at[g]], msgin, gsem).wait()

                @pl.when(g >= 1)
                def _wait_prev_scatter():
                    pltpu.make_async_copy(msgout, acc.at[db.at[g - 1]],
                                          ssem).wait()

                @plsc.parallel_loop(0, gpb, unroll=2)
                def _scale(jj):
                    w16v = wb[g, pl.ds(jj * 16, 16)]
                    for m in range(16):
                        wsp = _splat(w16v, m)
                        j = jj * 16 + m
                        for k in range(kch):
                            msgout[j, pl.ds(k * 16, 16)] = (
                                msgin[j, pl.ds(k * 16, 16)] * wsp)
                pltpu.async_copy(msgout, acc.at[db.at[g]], ssem, add=True)

                @pl.when(g + 1 < nb)
                def _next_gather():
                    pltpu.async_copy(src_p.at[sb.at[g + 1]], msgin, gsem)
                return _
            lax.fori_loop(0, nb, edge_block, 0)
            pltpu.make_async_copy(msgout, acc.at[db.at[nb - 1]],
                                  ssem).wait()
            plsc.subcore_barrier()

            if do_hist:
                def merge(t, _):
                    base = t * 128
                    for m in range(8):
                        idc[0, pl.ds(m * 16, 16)] = (_full16(base + m * 16)
                                                     + iota)

                    def stg(q, _):
                        hstage[q, pl.ds(0, 16)] = hist[pl.ds((base + q) * 16,
                                                             16)]
                        return _
                    lax.fori_loop(0, 128, stg, 0)
                    pltpu.sync_copy(hstage, cnt.at[idc.at[0]], add=True)
                    return _
                lax.fori_loop(0, r_hist // 128, merge, 0)
                plsc.subcore_barrier()

            out_p = outs[ph["out"]]
            prev_p = None if ph["prev"] is None else outs[ph["prev"]]
            relu = ph["relu"]

            def fin(i, _):
                r = row0 + 128 * i
                pltpu.sync_copy(acc.at[pl.ds(r, 128)], sumb)
                if inv_from_hbm:
                    pltpu.sync_copy(
                        invh.at[pl.ds(c * ndpad + r, 128)], invb.at[0])
                else:
                    pltpu.sync_copy(
                        cnt.at[pl.ds(lax.shift_right_logical(r, 4), 8)], cb)
                if prev_p is not None:
                    pltpu.sync_copy(prev_p.at[pl.ds(obase0 + 128 * i, 128)],
                                    ob)

                def rowfn(m, _):
                    q = lax.shift_right_logical(m, 4)
                    lane = lax.bitwise_and(m, 15)
                    if inv_from_hbm:
                        iv16 = invb[0, pl.ds(q * 16, 16)]
                        inv = _splat(iv16, lane)
                    else:
                        cv = cb[q, pl.ds(0, 16)]
                        cm = _splat(cv, lane)
                        inv = jnp.where(cm > 0.0,
                                        1.0 / jnp.maximum(cm, 1.0), 0.0)
                    for k in range(kch):
                        v = sumb[m, pl.ds(k * 16, 16)] * inv
                        if prev_p is not None:
                            v = v + ob[m, pl.ds(k * 16, 16)]
                        if relu:
                            v = jnp.maximum(v, 0.0)
                        ob[m, pl.ds(k * 16, 16)] = v
                    return _
                lax.fori_loop(0, 128, rowfn, 0)
                pltpu.sync_copy(ob, out_p.at[pl.ds(obase0 + 128 * i, 128)])
                return _
            lax.fori_loop(0, per_tile // 128, fin, 0)
            if p + 1 < len(phases):
                plsc.subcore_barrier()

    return seg


def _pad_edges(src, dst, w, epad, trash):
    padn = epad - src.shape[0]
    src = jnp.concatenate([src.astype(jnp.int32),
                           jnp.zeros((padn,), jnp.int32)])
    dst = jnp.concatenate([dst.astype(jnp.int32),
                           jnp.full((padn,), trash, jnp.int32)])
    w = jnp.concatenate([w, jnp.zeros((padn,), jnp.float32)])
    return src, dst, w


def kernel(h_w, h_t, h_d, ww_src, ww_dst, wt_src, wt_dst, wd_src, wd_dst,
           td_src, td_dst, tt_src, tt_dst, w_ww, w_wt, w_wd, w_td, w_tt,
           W_ww, b_ww, W_wt, b_wt, W_wd, b_wd, W_td, b_td, W_tt, b_tt):
    w_outs = _linear_chain(
        h_w,
        [W_ww, b_ww.reshape(1, D), W_wt, b_wt.reshape(1, D),
         W_wd, b_wd.reshape(1, D)],
        out_slices=[(0, 64), (64, 64)] + [(16 * k, 16) for k in range(8)],
        blk=1000)
    hw64 = w_outs[:2]
    hw16 = w_outs[2:]
    ht64 = _linear_chain(
        h_t, [W_td, b_td.reshape(1, D), W_tt, b_tt.reshape(1, D)],
        out_slices=[(0, 64), (64, 64)], blk=1000)

    epad, epad_w = 102400, 204800
    wt_s, wt_d, wt_w = _pad_edges(wt_src, wt_dst, w_wt, epad, NT)
    wd_s, wd_d, wd_w = _pad_edges(wd_src, wd_dst, w_wd, epad, ND)
    tt_s, tt_d, tt_w = _pad_edges(tt_src, tt_dst, w_tt, epad, NT)
    td_s, td_d, td_w = _pad_edges(td_src, td_dst, w_td, epad, ND)
    ww_s, ww_d, ww_w = _pad_edges(ww_src, ww_dst, w_ww, epad_w, NW)

    def slabs(x, nb, bl):
        return x.reshape(-1, nb, bl)

    ab_s = slabs(jnp.concatenate([wt_s, wd_s]), 20, 320)
    ab_d = slabs(jnp.concatenate([wt_d, wd_d]), 20, 320)
    ab_w = slabs(jnp.concatenate([wt_w, wd_w]), 20, 320)
    b_s = slabs(jnp.concatenate([tt_s, td_s]), 20, 320)
    b_d = slabs(jnp.concatenate([tt_d, td_d]), 20, 320)
    b_w = slabs(jnp.concatenate([tt_w, td_w]), 20, 320)
    ww_sb, ww_db, ww_wb = (slabs(ww_s, 32, 400), slabs(ww_d, 32, 400),
                           slabs(ww_w, 32, 400))

    counts_w = _make_counts(NWP, 32, 400)
    seg_ab = _make_seg(
        64, NDP, 20, 320, 16, 0, False, nsrc=4, nesets=2, nout=4,
        phases=[
            dict(src=0, eset=0, hist=True, prev=None, relu=False, out=0),
            dict(src=1, eset=0, hist=False, prev=None, relu=False, out=1),
            dict(src=2, eset=1, hist=True, prev=0, relu=True, out=2),
            dict(src=3, eset=1, hist=False, prev=1, relu=True, out=3),
        ])
    seg_w = _make_seg(
        16, NWP, 32, 400, 0, NW, True, nsrc=4, nesets=1, nout=4,
        phases=[dict(src=q, eset=0, hist=False, prev=None, relu=True, out=q)
                for q in range(4)])

    inv_ww = counts_w(ww_db)

    wsrcs = [jnp.concatenate([hw16[2 * q], hw16[2 * q + 1]])
             for q in range(4)]
    _, _, out_b0, out_b1 = seg_ab(
        hw64[0], hw64[1], ht64[0], ht64[1],
        ab_s, ab_d, ab_w, b_s, b_d, b_w)
    out_w = seg_w(wsrcs[0], wsrcs[1], wsrcs[2], wsrcs[3],
                  ww_sb, ww_db, ww_wb, inv_ww)

    new_w = jnp.concatenate(
        [piece for p in range(4)
         for piece in (out_w[p][:NW], out_w[p][NWP:NWP + NW])], axis=1)
    new_t = jnp.concatenate([out_b0[:NT], out_b1[:NT]], axis=1)
    new_d = jnp.concatenate([out_b0[NDP:NDP + ND], out_b1[NDP:NDP + ND]],
                            axis=1)
    return (new_w, new_t, new_d)


# speedup vs baseline: 1.0539x; 1.0539x over previous
import functools

import jax
import jax.numpy as jnp
from jax import lax
from jax.experimental import pallas as pl
from jax.experimental.pallas import tpu as pltpu
from jax.experimental.pallas import tpu_sc as plsc

NW, NT, ND, D = 50000, 10000, 10000, 128
E_WW, E = 200000, 100000
LANES = 16
NDP = 10240
NWP = 51200


def _linear_chain_kernel(nmm, slices, x_ref, *refs):
    ws = refs[:2 * nmm]
    outs = refs[2 * nmm:]
    y = x_ref[...]
    for i in range(nmm):
        w = ws[2 * i][...]
        b = ws[2 * i + 1][...]
        y = lax.dot_general(y, w, (((1,), (1,)), ((), ())),
                            preferred_element_type=jnp.float32) + b
    for r, (start, nc) in zip(outs, slices):
        r[...] = y[:, start:start + nc]


def _linear_chain(x, wbs, out_slices, blk):
    n = x.shape[0]
    nmm = len(wbs) // 2
    grid = (n // blk,)
    w_spec = pl.BlockSpec((D, D), lambda i: (0, 0))
    b_spec = pl.BlockSpec((1, D), lambda i: (0, 0))
    in_specs = [pl.BlockSpec((blk, D), lambda i: (i, 0))]
    for _ in range(nmm):
        in_specs += [w_spec, b_spec]
    out_shapes = [jax.ShapeDtypeStruct((n, c), jnp.float32)
                  for _, c in out_slices]
    out_specs = [pl.BlockSpec((blk, c), lambda i: (i, 0))
                 for _, c in out_slices]
    return pl.pallas_call(
        functools.partial(_linear_chain_kernel, nmm, out_slices),
        grid=grid,
        in_specs=in_specs,
        out_specs=out_specs,
        out_shape=out_shapes,
    )(x, *wbs)


_SC_PARAMS = pltpu.CompilerParams(needs_layout_passes=False,
                                  use_tc_tiling_on_sc=False)


def _full16(v, dtype=jnp.int32):
    return jnp.full((LANES,), v, dtype=dtype)


def _splat(vec, m):
    return jnp.take_along_axis(vec, _full16(m), axis=0)


def _make_counts(ndpad, nb, bl):
    r_hist = ndpad // 16
    per_tile = ndpad // 16
    gpb = bl // 16
    mesh = plsc.VectorSubcoreMesh(core_axis_name="c", subcore_axis_name="s")
    scratch = [
        pltpu.VMEM((nb, bl), jnp.int32),
        pltpu.VMEM((r_hist * 16,), jnp.float32),
        pltpu.VMEM((128, 16), jnp.float32),
        pltpu.VMEM((1, 128), jnp.int32),
        pltpu.VMEM((8, 16), jnp.float32),
        pltpu.VMEM((1, 128), jnp.float32),
        pltpu.VMEM_SHARED((r_hist, 16), jnp.float32),
    ]
    out_type = jax.ShapeDtypeStruct((2 * ndpad,), jnp.float32)

    @functools.partial(pl.kernel, out_type=out_type, mesh=mesh,
                       scratch_types=scratch, compiler_params=_SC_PARAMS)
    def counts(didx, out, db, hist, hstage, idc, cb, invs, cnt):
        c = lax.axis_index("c")
        s = lax.axis_index("s")
        iota = lax.iota(jnp.int32, LANES)
        ones = jnp.full((LANES,), 1.0, jnp.float32)
        zeros = jnp.zeros((LANES,), jnp.float32)

        pltpu.sync_copy(didx.at[s], db)

        def zh(r, _):
            hist[pl.ds(r * 16, 16)] = zeros
            return _
        lax.fori_loop(0, r_hist, zh, 0)
        for r in range(8):
            cb[r, pl.ds(0, 16)] = zeros
        crow0 = s * (r_hist // 16)

        def zc(i, _):
            pltpu.sync_copy(cb, cnt.at[pl.ds(crow0 + 8 * i, 8)])
            return _
        lax.fori_loop(0, r_hist // 128, zc, 0)
        plsc.subcore_barrier()

        def per_group(g, _):
            def pg2(jj, _):
                d16v = db[g, pl.ds(jj * 16, 16)]
                for m in range(16):
                    dsp = _splat(d16v, m)
                    plsc.addupdate_scatter(hist, [dsp], ones, mask=iota == 0)
                return _
            lax.fori_loop(0, gpb, pg2, 0)
            return _
        lax.fori_loop(0, nb, per_group, 0)
        plsc.subcore_barrier()

        def merge(t, _):
            base = t * 128
            for m in range(8):
                idc[0, pl.ds(m * 16, 16)] = _full16(base + m * 16) + iota

            def stg(q, _):
                hstage[q, pl.ds(0, 16)] = hist[pl.ds((base + q) * 16, 16)]
                return _
            lax.fori_loop(0, 128, stg, 0)
            pltpu.sync_copy(hstage, cnt.at[idc.at[0]], add=True)
            return _
        lax.fori_loop(0, r_hist // 128, merge, 0)
        plsc.subcore_barrier()

        row0 = s * per_tile
        obase = c * ndpad + row0

        def fin(i, _):
            r = row0 + 128 * i
            pltpu.sync_copy(cnt.at[pl.ds(lax.shift_right_logical(r, 4), 8)],
                            cb)
            for q in range(8):
                cv = cb[q, pl.ds(0, 16)]
                inv = jnp.where(cv > 0.0, 1.0 / jnp.maximum(cv, 1.0), 0.0)
                invs[0, pl.ds(q * 16, 16)] = inv
            pltpu.sync_copy(invs.at[0], out.at[pl.ds(obase + 128 * i, 128)])
            return _
        lax.fori_loop(0, per_tile // 128, fin, 0)

    return counts


def _make_seg(drow, ndpad, nb, bl, tile_stride, src_off, inv_from_hbm,
              nsrc, nesets, nout, phases):
    r_hist = ndpad // 16
    per_tile = ndpad // 16
    assert per_tile % 128 == 0
    mesh = plsc.VectorSubcoreMesh(core_axis_name="c", subcore_axis_name="s")
    kch = drow // 16
    has_prev = any(ph["prev"] is not None for ph in phases)

    gpb = bl // 16
    scratch = [
        pltpu.VMEM((nb, bl), jnp.int32),
        pltpu.VMEM((nb, bl), jnp.int32),
        pltpu.VMEM((nb, bl), jnp.float32),
        pltpu.VMEM((2, bl, drow), jnp.float32),
        pltpu.VMEM((2, bl, drow), jnp.float32),
        pltpu.VMEM((128, drow), jnp.float32),
        pltpu.VMEM((128, drow), jnp.float32),
        pltpu.SemaphoreType.DMA,
        pltpu.SemaphoreType.DMA,
        pltpu.SemaphoreType.DMA,
        pltpu.SemaphoreType.DMA,
    ]
    if inv_from_hbm:
        scratch.append(pltpu.VMEM((1, 128), jnp.float32))
    else:
        scratch += [
            pltpu.VMEM((r_hist * 16,), jnp.float32),
            pltpu.VMEM((128, 16), jnp.float32),
            pltpu.VMEM((1, 128), jnp.int32),
            pltpu.VMEM((8, 16), jnp.float32),
        ]
    scratch.append(pltpu.VMEM_SHARED((ndpad, drow), jnp.float32))
    if not inv_from_hbm:
        scratch.append(pltpu.VMEM_SHARED((r_hist, 16), jnp.float32))

    out_type = [jax.ShapeDtypeStruct((2 * ndpad, drow), jnp.float32)
                for _ in range(nout)]

    @functools.partial(pl.kernel, out_type=out_type, mesh=mesh,
                       scratch_types=scratch, compiler_params=_SC_PARAMS)
    def seg(*args):
        it = iter(args)
        srcs = [next(it) for _ in range(nsrc)]
        esets = [(next(it), next(it), next(it)) for _ in range(nesets)]
        invh = next(it) if inv_from_hbm else None
        outs = [next(it) for _ in range(nout)]
        sb, db, wb, msgin, msgout, sumb, ob = (next(it) for _ in range(7))
        gsems = [next(it), next(it)]
        ssems = [next(it), next(it)]
        if inv_from_hbm:
            invb = next(it)
        else:
            hist, hstage, idc, cb = (next(it) for _ in range(4))
        acc = next(it)
        cnt = None if inv_from_hbm else next(it)

        c = lax.axis_index("c")
        s = lax.axis_index("s")
        iota = lax.iota(jnp.int32, LANES)
        ones = jnp.full((LANES,), 1.0, jnp.float32)
        zeros = jnp.zeros((LANES,), jnp.float32)

        tid = c * tile_stride + s
        row0 = s * per_tile
        obase0 = c * ndpad + row0

        def preload(eset):
            sidx, didx, ew = esets[eset]
            pltpu.sync_copy(sidx.at[tid], sb)
            pltpu.sync_copy(didx.at[tid], db)
            pltpu.sync_copy(ew.at[tid], wb)
            if src_off:
                off = _full16(c * src_off)

                def adj(g, _):
                    def adj2(k, _):
                        sb[g, pl.ds(k * 16, 16)] = (sb[g, pl.ds(k * 16, 16)]
                                                    + off)
                        return _
                    lax.fori_loop(0, gpb, adj2, 0)
                    return _
                lax.fori_loop(0, nb, adj, 0)

        cur_eset = None
        for p, ph in enumerate(phases):
            do_hist = (not inv_from_hbm) and ph["hist"]
            if ph["eset"] != cur_eset:
                preload(ph["eset"])
                cur_eset = ph["eset"]

            def zz(r, _):
                for k in range(kch):
                    sumb[r, pl.ds(k * 16, 16)] = zeros
                return _
            lax.fori_loop(0, 128, zz, 0)

            def za(i, _):
                pltpu.sync_copy(sumb, acc.at[pl.ds(row0 + 128 * i, 128)])
                return _
            lax.fori_loop(0, per_tile // 128, za, 0)
            if do_hist:
                def zh(r, _):
                    hist[pl.ds(r * 16, 16)] = zeros
                    return _
                lax.fori_loop(0, r_hist, zh, 0)
                for r in range(8):
                    cb[r, pl.ds(0, 16)] = zeros
                crow0 = s * (r_hist // 16)

                def zc(i, _):
                    pltpu.sync_copy(cb, cnt.at[pl.ds(crow0 + 8 * i, 8)])
                    return _
                lax.fori_loop(0, r_hist // 128, zc, 0)
            plsc.subcore_barrier()

            if do_hist:
                def histloop(g, _):
                    def histg(jj, _):
                        d16v = db[g, pl.ds(jj * 16, 16)]
                        for m in range(16):
                            dsp = _splat(d16v, m)
                            plsc.addupdate_scatter(hist, [dsp], ones,
                                                   mask=iota == 0)
                        return _
                    lax.fori_loop(0, gpb, histg, 0)
                    return _
                lax.fori_loop(0, nb, histloop, 0)

            src_p = srcs[ph["src"]]
            pltpu.async_copy(src_p.at[sb.at[0]], msgin.at[0], gsems[0])
            pltpu.async_copy(src_p.at[sb.at[1]], msgin.at[1], gsems[1])

            def edge_pair(h, _):
                for b in range(2):
                    g = 2 * h + b
                    pltpu.make_async_copy(src_p.at[sb.at[g]], msgin.at[b],
                                          gsems[b]).wait()

                    @pl.when(g >= 2)
                    def _wait_prev_scatter():
                        pltpu.make_async_copy(msgout.at[b],
                                              acc.at[db.at[g - 2]],
                                              ssems[b]).wait()

                    @plsc.parallel_loop(0, gpb, unroll=2)
                    def _scale(jj):
                        w16v = wb[g, pl.ds(jj * 16, 16)]
                        for m in range(16):
                            wsp = _splat(w16v, m)
                            j = jj * 16 + m
                            for k in range(kch):
                                msgout[b, j, pl.ds(k * 16, 16)] = (
                                    msgin[b, j, pl.ds(k * 16, 16)] * wsp)
                    pltpu.async_copy(msgout.at[b], acc.at[db.at[g]],
                                     ssems[b], add=True)

                    @pl.when(g + 2 < nb)
                    def _next_gather():
                        pltpu.async_copy(src_p.at[sb.at[g + 2]], msgin.at[b],
                                         gsems[b])
                return _
            lax.fori_loop(0, nb // 2, edge_pair, 0)
            for b in range(2):
                pltpu.make_async_copy(msgout.at[b], acc.at[db.at[nb - 2 + b]],
                                      ssems[b]).wait()
            plsc.subcore_barrier()

            if do_hist:
                def merge(t, _):
                    base = t * 128
                    for m in range(8):
                        idc[0, pl.ds(m * 16, 16)] = (_full16(base + m * 16)
                                                     + iota)

                    def stg(q, _):
                        hstage[q, pl.ds(0, 16)] = hist[pl.ds((base + q) * 16,
                                                             16)]
                        return _
                    lax.fori_loop(0, 128, stg, 0)
                    pltpu.sync_copy(hstage, cnt.at[idc.at[0]], add=True)
                    return _
                lax.fori_loop(0, r_hist // 128, merge, 0)
                plsc.subcore_barrier()

            out_p = outs[ph["out"]]
            prev_p = None if ph["prev"] is None else outs[ph["prev"]]
            relu = ph["relu"]

            def fin(i, _):
                r = row0 + 128 * i
                pltpu.sync_copy(acc.at[pl.ds(r, 128)], sumb)
                if inv_from_hbm:
                    pltpu.sync_copy(
                        invh.at[pl.ds(c * ndpad + r, 128)], invb.at[0])
                else:
                    pltpu.sync_copy(
                        cnt.at[pl.ds(lax.shift_right_logical(r, 4), 8)], cb)
                if prev_p is not None:
                    pltpu.sync_copy(prev_p.at[pl.ds(obase0 + 128 * i, 128)],
                                    ob)

                def rowfn(m, _):
                    q = lax.shift_right_logical(m, 4)
                    lane = lax.bitwise_and(m, 15)
                    if inv_from_hbm:
                        iv16 = invb[0, pl.ds(q * 16, 16)]
                        inv = _splat(iv16, lane)
                    else:
                        cv = cb[q, pl.ds(0, 16)]
                        cm = _splat(cv, lane)
                        inv = jnp.where(cm > 0.0,
                                        1.0 / jnp.maximum(cm, 1.0), 0.0)
                    for k in range(kch):
                        v = sumb[m, pl.ds(k * 16, 16)] * inv
                        if prev_p is not None:
                            v = v + ob[m, pl.ds(k * 16, 16)]
                        if relu:
                            v = jnp.maximum(v, 0.0)
                        ob[m, pl.ds(k * 16, 16)] = v
                    return _
                lax.fori_loop(0, 128, rowfn, 0)
                pltpu.sync_copy(ob, out_p.at[pl.ds(obase0 + 128 * i, 128)])
                return _
            lax.fori_loop(0, per_tile // 128, fin, 0)
            if p + 1 < len(phases):
                plsc.subcore_barrier()

    return seg


def _pad_edges(src, dst, w, epad, trash):
    padn = epad - src.shape[0]
    src = jnp.concatenate([src.astype(jnp.int32),
                           jnp.zeros((padn,), jnp.int32)])
    dst = jnp.concatenate([dst.astype(jnp.int32),
                           jnp.full((padn,), trash, jnp.int32)])
    w = jnp.concatenate([w, jnp.zeros((padn,), jnp.float32)])
    return src, dst, w


def kernel(h_w, h_t, h_d, ww_src, ww_dst, wt_src, wt_dst, wd_src, wd_dst,
           td_src, td_dst, tt_src, tt_dst, w_ww, w_wt, w_wd, w_td, w_tt,
           W_ww, b_ww, W_wt, b_wt, W_wd, b_wd, W_td, b_td, W_tt, b_tt):
    w_outs = _linear_chain(
        h_w,
        [W_ww, b_ww.reshape(1, D), W_wt, b_wt.reshape(1, D),
         W_wd, b_wd.reshape(1, D)],
        out_slices=[(0, 64), (64, 64)] + [(16 * k, 16) for k in range(8)],
        blk=1000)
    hw64 = w_outs[:2]
    hw16 = w_outs[2:]
    ht64 = _linear_chain(
        h_t, [W_td, b_td.reshape(1, D), W_tt, b_tt.reshape(1, D)],
        out_slices=[(0, 64), (64, 64)], blk=1000)

    epad, epad_w = 102400, 204800
    wt_s, wt_d, wt_w = _pad_edges(wt_src, wt_dst, w_wt, epad, NT)
    wd_s, wd_d, wd_w = _pad_edges(wd_src, wd_dst, w_wd, epad, ND)
    tt_s, tt_d, tt_w = _pad_edges(tt_src, tt_dst, w_tt, epad, NT)
    td_s, td_d, td_w = _pad_edges(td_src, td_dst, w_td, epad, ND)
    ww_s, ww_d, ww_w = _pad_edges(ww_src, ww_dst, w_ww, epad_w, NW)

    def slabs(x, nb, bl):
        return x.reshape(-1, nb, bl)

    ab_s = slabs(jnp.concatenate([wt_s, wd_s]), 40, 160)
    ab_d = slabs(jnp.concatenate([wt_d, wd_d]), 40, 160)
    ab_w = slabs(jnp.concatenate([wt_w, wd_w]), 40, 160)
    b_s = slabs(jnp.concatenate([tt_s, td_s]), 40, 160)
    b_d = slabs(jnp.concatenate([tt_d, td_d]), 40, 160)
    b_w = slabs(jnp.concatenate([tt_w, td_w]), 40, 160)
    ww_sb, ww_db, ww_wb = (slabs(ww_s, 50, 256), slabs(ww_d, 50, 256),
                           slabs(ww_w, 50, 256))

    counts_w = _make_counts(NWP, 50, 256)
    seg_ab = _make_seg(
        64, NDP, 40, 160, 16, 0, False, nsrc=4, nesets=2, nout=4,
        phases=[
            dict(src=0, eset=0, hist=True, prev=None, relu=False, out=0),
            dict(src=1, eset=0, hist=False, prev=None, relu=False, out=1),
            dict(src=2, eset=1, hist=True, prev=0, relu=True, out=2),
            dict(src=3, eset=1, hist=False, prev=1, relu=True, out=3),
        ])
    seg_w = _make_seg(
        16, NWP, 50, 256, 0, NW, True, nsrc=4, nesets=1, nout=4,
        phases=[dict(src=q, eset=0, hist=False, prev=None, relu=True, out=q)
                for q in range(4)])

    inv_ww = counts_w(ww_db)

    wsrcs = [jnp.concatenate([hw16[2 * q], hw16[2 * q + 1]])
             for q in range(4)]
    _, _, out_b0, out_b1 = seg_ab(
        hw64[0], hw64[1], ht64[0], ht64[1],
        ab_s, ab_d, ab_w, b_s, b_d, b_w)
    out_w = seg_w(wsrcs[0], wsrcs[1], wsrcs[2], wsrcs[3],
                  ww_sb, ww_db, ww_wb, inv_ww)

    new_w = jnp.concatenate(
        [piece for p in range(4)
         for piece in (out_w[p][:NW], out_w[p][NWP:NWP + NW])], axis=1)
    new_t = jnp.concatenate([out_b0[:NT], out_b1[:NT]], axis=1)
    new_d = jnp.concatenate([out_b0[NDP:NDP + ND], out_b1[NDP:NDP + ND]],
                            axis=1)
    return (new_w, new_t, new_d)
